# baseline (speedup 1.0000x reference)
import functools

import jax
import jax.numpy as jnp
from jax import lax
from jax.experimental import pallas as pl
from jax.experimental.pallas import tpu as pltpu
from jax.experimental.pallas import tpu_sc as plsc

BATCH = 1024
OUT_FEATURES = 64

_info = plsc.get_sparse_core_info()
_NC = _info.num_cores
_NS = _info.num_subcores
_L = _info.num_lanes
_NW = _NC * _NS
_B_PER_W = BATCH // _NW


@functools.partial(
    pl.kernel,
    mesh=plsc.VectorSubcoreMesh(core_axis_name="c", subcore_axis_name="s"),
    out_type=jax.ShapeDtypeStruct((BATCH, OUT_FEATURES), jnp.float32),
    scratch_types=[
        pltpu.VMEM((_B_PER_W, 1), jnp.float32),
        pltpu.VMEM((_B_PER_W, OUT_FEATURES), jnp.float32),
        pltpu.SemaphoreType.DMA,
    ],
    compiler_params=pltpu.CompilerParams(needs_layout_passes=False),
)
def _sc_gather(table_hbm, xf_hbm, out_hbm, xf_v, rows_v, sem):
    wid = lax.axis_index("s") * _NC + lax.axis_index("c")
    base = wid * _B_PER_W
    pltpu.sync_copy(xf_hbm.at[pl.ds(base, _B_PER_W)], xf_v)
    lane = lax.iota(jnp.int32, _L)
    col0 = jnp.zeros((_L,), jnp.int32)
    copies = []
    for j in range(_B_PER_W // _L):
        chunk_f = plsc.load_gather(xf_v, [lane + j * _L, col0])
        chunk = chunk_f.astype(jnp.int32)
        for i in range(_L):
            r = jnp.squeeze(lax.slice(chunk, (i,), (i + 1,)))
            c = pltpu.async_copy(table_hbm.at[r], rows_v.at[j * _L + i], sem)
            copies.append(c)
    for c in copies:
        c.wait()
    pltpu.sync_copy(rows_v, out_hbm.at[pl.ds(base, _B_PER_W)])


def kernel(x, feature_array):
    return _sc_gather(feature_array, x)
